# baseline (speedup 1.0000x reference)
import functools

import jax
import jax.numpy as jnp
from jax import lax
from jax.experimental import pallas as pl
from jax.experimental.pallas import tpu as pltpu
from jax.experimental.pallas import tpu_sc as plsc

NUM_SAMPLES = 100000
NUM_FEATURES = 64
NUM_CLUSTERS = 2000
C_PAD = 2048
BATCH = 1024
TEMP = 0.05

NC = 2
NS = 16
NW = NC * NS
NWIDE = NUM_SAMPLES // 2
CHUNK = 400
NCHUNK = NWIDE // CHUNK
SUB = 80
NSUB = CHUNK // SUB
QPW = BATCH // NW


def _sc_body(feat2_hbm, leo_hbm, labflat_hbm, idxflat_hbm, zb_hbm, zc_hbm,
             ones_hbm, bank_out, cnt_out, tgt_out,
             idxe_v, idxo_v, evn_v, odd_v, ones_v, qidx_v, tgt_v, bank_sh, cnt_sh,
             lsem, ssem, gsem):
    cid = lax.axis_index("c")
    sid = lax.axis_index("s")
    wid = sid * NC + cid

    def load(t, b):
        j = wid + NW * t
        re = pltpu.async_copy(feat2_hbm.at[pl.ds(j * CHUNK, CHUNK),
                                           0:NUM_FEATURES],
                              evn_v.at[b], lsem[b])
        ro = pltpu.async_copy(feat2_hbm.at[pl.ds(j * CHUNK, CHUNK),
                                           NUM_FEATURES:2 * NUM_FEATURES],
                              odd_v.at[b], lsem[b])
        e = pltpu.async_copy(leo_hbm.at[0, pl.ds(j * CHUNK, CHUNK)],
                             idxe_v.at[b], lsem[b])
        o = pltpu.async_copy(leo_hbm.at[1, pl.ds(j * CHUNK, CHUNK)],
                             idxo_v.at[b], lsem[b])
        return (re, ro, e, o)

    ld = [None, None]
    ld[0] = load(0, 0)

    @pl.when(sid == 0)
    def _():
        pltpu.sync_copy(zb_hbm, bank_sh)
        pltpu.sync_copy(zc_hbm, cnt_sh)

    pltpu.sync_copy(ones_hbm, ones_v)
    pltpu.sync_copy(idxflat_hbm.at[pl.ds(wid * QPW, QPW)], qidx_v)
    pltpu.async_copy(labflat_hbm.at[qidx_v], tgt_v, gsem).wait()
    pltpu.sync_copy(tgt_v, tgt_out.at[pl.ds(wid * QPW, QPW)])

    plsc.subcore_barrier()

    has_t3 = wid < NCHUNK - 3 * NW

    def fire_scatters(b):
        out = []
        for r in range(NSUB):
            s = pl.ds(r * SUB, SUB)
            out.append(
                pltpu.async_copy(evn_v.at[b, s, :],
                                 bank_sh.at[idxe_v.at[b, s]], ssem[b],
                                 add=True))
            out.append(
                pltpu.async_copy(odd_v.at[b, s, :],
                                 bank_sh.at[idxo_v.at[b, s]], ssem[b],
                                 add=True))
            out.append(
                pltpu.async_copy(ones_v, cnt_sh.at[idxe_v.at[b, s]],
                                 ssem[b], add=True))
            out.append(
                pltpu.async_copy(ones_v, cnt_sh.at[idxo_v.at[b, s]],
                                 ssem[b], add=True))
        return out

    scat = [[], []]
    for t in range(3):
        b = t & 1
        for d in ld[b]:
            d.wait()
        scat[b] = fire_scatters(b)
        nb = 1 - b
        for d in scat[nb]:
            d.wait()
        scat[nb] = []
        if t < 2:
            ld[nb] = load(t + 1, nb)
        else:
            @pl.when(has_t3)
            def _():
                ld3 = load(3, nb)
                for d in ld3:
                    d.wait()
                for d in fire_scatters(nb):
                    d.wait()

    for d in scat[0]:
        d.wait()

    plsc.subcore_barrier()

    @pl.when(sid == 0)
    def _():
        pltpu.sync_copy(bank_sh, bank_out.at[cid])
        pltpu.sync_copy(cnt_sh, cnt_out.at[cid])


_sc_call = functools.partial(
    pl.kernel,
    out_type=(
        jax.ShapeDtypeStruct((NC, C_PAD, NUM_FEATURES), jnp.float32),
        jax.ShapeDtypeStruct((NC, C_PAD, 16), jnp.float32),
        jax.ShapeDtypeStruct((BATCH,), jnp.int32),
    ),
    mesh=plsc.VectorSubcoreMesh(core_axis_name="c", subcore_axis_name="s"),
    compiler_params=pltpu.CompilerParams(use_tc_tiling_on_sc=False),
    scratch_types=(
        pltpu.VMEM((2, CHUNK), jnp.int32),
        pltpu.VMEM((2, CHUNK), jnp.int32),
        pltpu.VMEM((2, CHUNK, NUM_FEATURES), jnp.float32),
        pltpu.VMEM((2, CHUNK, NUM_FEATURES), jnp.float32),
        pltpu.VMEM((SUB, 16), jnp.float32),
        pltpu.VMEM((QPW,), jnp.int32),
        pltpu.VMEM((QPW,), jnp.int32),
        pltpu.VMEM_SHARED((C_PAD, NUM_FEATURES), jnp.float32),
        pltpu.VMEM_SHARED((C_PAD, 16), jnp.float32),
        (pltpu.SemaphoreType.DMA, pltpu.SemaphoreType.DMA),
        (pltpu.SemaphoreType.DMA, pltpu.SemaphoreType.DMA),
        pltpu.SemaphoreType.DMA,
    ),
)(_sc_body)


def _tc_body(x_ref, bank_ref, cnt_ref, tgt_ref, out_ref):
    x = x_ref[...]
    bank = bank_ref[0] + bank_ref[1]
    cnt = cnt_ref[0, :, 0:1] + cnt_ref[1, :, 0:1]
    dots = lax.dot_general(x, bank, (((1,), (1,)), ((), ())),
                           preferred_element_type=jnp.float32,
                           precision=lax.Precision.HIGHEST)
    denom = jnp.where(cnt > 0.0, cnt, 1.0)
    scale = (1.0 / TEMP) / denom
    vec = dots * scale.T
    mask = (cnt > 0.0).astype(jnp.float32).T
    exps = jnp.exp(vec) * mask
    sums = jnp.sum(exps, axis=1, keepdims=True) + 1e-6
    cids = lax.broadcasted_iota(jnp.int32, exps.shape, 1)
    texp = jnp.sum(jnp.where(cids == tgt_ref[...], exps, 0.0),
                   axis=1, keepdims=True)
    logp = jnp.log(texp / sums + 1e-6)
    out_ref[...] = -jnp.sum(logp, axis=0, keepdims=True) / float(BATCH)


_tc_call = pl.pallas_call(
    _tc_body,
    out_shape=jax.ShapeDtypeStruct((1, 1), jnp.float32),
)


def kernel(inputs, indexes, features, label):
    feat2 = features.reshape(NWIDE, 2 * NUM_FEATURES)
    leo = label.reshape(NWIDE, 2).T
    zb = jnp.zeros((C_PAD, NUM_FEATURES), jnp.float32)
    zc = jnp.zeros((C_PAD, 16), jnp.float32)
    ones = jnp.ones((SUB, 16), jnp.float32)
    bank2, cnt2, tgt = _sc_call(feat2, leo, label, indexes, zb, zc, ones)
    loss = _tc_call(inputs, bank2, cnt2, tgt.reshape(BATCH, 1))
    return loss.reshape(())


# speedup vs baseline: 1.2698x; 1.2698x over previous
import functools

import jax
import jax.numpy as jnp
from jax import lax
from jax.experimental import pallas as pl
from jax.experimental.pallas import tpu as pltpu
from jax.experimental.pallas import tpu_sc as plsc

NUM_SAMPLES = 100000
NUM_FEATURES = 64
NUM_CLUSTERS = 2000
C_PAD = 2048
BATCH = 1024
TEMP = 0.05

NC = 2
NS = 16
NW = NC * NS
CHUNK = 800
NCHUNK = NUM_SAMPLES // CHUNK
SUB = 80
NSUB = CHUNK // SUB
QPW = BATCH // NW


def _sc_body(feat_hbm, labflat_hbm, idxflat_hbm, bank_out, cnt_out, tgt_out,
             idx_v, rows_v, ones_v, zb_v, zc_v, qidx_v, tgt_v, bank_sh, cnt_sh,
             lsem, ssem, gsem):
    cid = lax.axis_index("c")
    sid = lax.axis_index("s")
    wid = sid * NC + cid

    def load(t, b):
        j = wid + NW * t
        r = pltpu.async_copy(feat_hbm.at[pl.ds(j * CHUNK, CHUNK), :],
                             rows_v.at[b], lsem[b])
        i = pltpu.async_copy(labflat_hbm.at[pl.ds(j * CHUNK, CHUNK)],
                             idx_v.at[b], lsem[b])
        return (r, i)

    ld = [None, None]
    ld[0] = load(0, 0)

    zv = jnp.zeros((16,), jnp.float32)
    ov = jnp.ones((16,), jnp.float32)

    def _fill_z(r, _):
        for k in range(NUM_FEATURES // 16):
            zb_v[r, pl.ds(k * 16, 16)] = zv
        zc_v[r, pl.ds(0, 16)] = zv
        return 0

    lax.fori_loop(0, C_PAD // NS, _fill_z, 0)
    for r in range(SUB):
        ones_v[r, pl.ds(0, 16)] = ov
    stripe = pl.ds(sid * (C_PAD // NS), C_PAD // NS)
    pltpu.sync_copy(zb_v, bank_sh.at[stripe, :])
    pltpu.sync_copy(zc_v, cnt_sh.at[stripe, :])

    pltpu.sync_copy(idxflat_hbm.at[pl.ds(wid * QPW, QPW)], qidx_v)
    pltpu.async_copy(labflat_hbm.at[qidx_v], tgt_v, gsem).wait()
    pltpu.sync_copy(tgt_v, tgt_out.at[pl.ds(wid * QPW, QPW)])

    plsc.subcore_barrier()

    has_t3 = wid < NCHUNK - 3 * NW

    def fire_scatters(b):
        out = []
        for r in range(NSUB):
            s = pl.ds(r * SUB, SUB)
            out.append(
                pltpu.async_copy(rows_v.at[b, s, :],
                                 bank_sh.at[idx_v.at[b, s]], ssem[b],
                                 add=True))
            out.append(
                pltpu.async_copy(ones_v, cnt_sh.at[idx_v.at[b, s]],
                                 ssem[b], add=True))
        return out

    scat = [[], []]
    for t in range(3):
        b = t & 1
        for d in ld[b]:
            d.wait()
        scat[b] = fire_scatters(b)
        nb = 1 - b
        for d in scat[nb]:
            d.wait()
        scat[nb] = []
        if t < 2:
            ld[nb] = load(t + 1, nb)
        else:
            @pl.when(has_t3)
            def _():
                ld3 = load(3, nb)
                for d in ld3:
                    d.wait()
                for d in fire_scatters(nb):
                    d.wait()

    for d in scat[0]:
        d.wait()

    plsc.subcore_barrier()

    @pl.when(sid == 0)
    def _():
        pltpu.sync_copy(bank_sh, bank_out.at[cid])
        pltpu.sync_copy(cnt_sh, cnt_out.at[cid])


_sc_call = functools.partial(
    pl.kernel,
    out_type=(
        jax.ShapeDtypeStruct((NC, C_PAD, NUM_FEATURES), jnp.float32),
        jax.ShapeDtypeStruct((NC, C_PAD, 16), jnp.float32),
        jax.ShapeDtypeStruct((BATCH,), jnp.int32),
    ),
    mesh=plsc.VectorSubcoreMesh(core_axis_name="c", subcore_axis_name="s"),
    compiler_params=pltpu.CompilerParams(use_tc_tiling_on_sc=False),
    scratch_types=(
        pltpu.VMEM((2, CHUNK), jnp.int32),
        pltpu.VMEM((2, CHUNK, NUM_FEATURES), jnp.float32),
        pltpu.VMEM((SUB, 16), jnp.float32),
        pltpu.VMEM((C_PAD // NS, NUM_FEATURES), jnp.float32),
        pltpu.VMEM((C_PAD // NS, 16), jnp.float32),
        pltpu.VMEM((QPW,), jnp.int32),
        pltpu.VMEM((QPW,), jnp.int32),
        pltpu.VMEM_SHARED((C_PAD, NUM_FEATURES), jnp.float32),
        pltpu.VMEM_SHARED((C_PAD, 16), jnp.float32),
        (pltpu.SemaphoreType.DMA, pltpu.SemaphoreType.DMA),
        (pltpu.SemaphoreType.DMA, pltpu.SemaphoreType.DMA),
        pltpu.SemaphoreType.DMA,
    ),
)(_sc_body)


def _tc_body(x_ref, bank_ref, cnt_ref, tgt_ref, out_ref):
    x = x_ref[...]
    bank = bank_ref[0] + bank_ref[1]
    cnt = cnt_ref[0, :, 0:1] + cnt_ref[1, :, 0:1]
    dots = lax.dot_general(x, bank, (((0,), (1,)), ((), ())),
                           preferred_element_type=jnp.float32,
                           precision=lax.Precision.HIGHEST)
    denom = jnp.where(cnt > 0.0, cnt, 1.0)
    scale = (1.0 / TEMP) / denom
    vec = dots * scale.T
    mask = (cnt > 0.0).astype(jnp.float32).T
    exps = jnp.exp(vec) * mask
    sums = jnp.sum(exps, axis=1, keepdims=True) + 1e-6
    cids = lax.broadcasted_iota(jnp.int32, exps.shape, 1)
    texp = jnp.sum(jnp.where(cids == tgt_ref[...], exps, 0.0),
                   axis=1, keepdims=True)
    logp = jnp.log(texp / sums + 1e-6)
    out_ref[...] = -jnp.sum(logp, axis=0, keepdims=True) / float(BATCH)


_tc_call = pl.pallas_call(
    _tc_body,
    out_shape=jax.ShapeDtypeStruct((1, 1), jnp.float32),
)


def kernel(inputs, indexes, features, label):
    bank2, cnt2, tgt = _sc_call(features, label, indexes)
    loss = _tc_call(inputs.T, bank2, cnt2, tgt.reshape(BATCH, 1))
    return loss.reshape(())


# speedup vs baseline: 1.5190x; 1.1963x over previous
import functools

import jax
import jax.numpy as jnp
from jax import lax
from jax.experimental import pallas as pl
from jax.experimental.pallas import tpu as pltpu
from jax.experimental.pallas import tpu_sc as plsc

NUM_SAMPLES = 100000
NUM_FEATURES = 64
NF2 = 2 * NUM_FEATURES
NUM_CLUSTERS = 2000
C_PAD = 2048
BATCH = 1024
TEMP = 0.05

NC = 2
NS = 16
NW = NC * NS

RBLK = 2048
RHALF = RBLK // 2
NBLK = 48
MAIN = NBLK * RBLK
REM = NUM_SAMPLES - MAIN
REMH = REM // 2
NWIDE = (NBLK + 1) * RHALF

CW = 256
NCHUNK = NBLK * RHALF // CW
CPB = RHALF // CW
SUB = 128
NSUB = CW // SUB
QPW = BATCH // NW

TAIL = {0: ((0, 224), (224, 200)), 1: ((424, 224), (648, 200))}
TAIL_SUBS = {224: (128, 96), 200: (128, 72)}


def _rp_body(xt_ref, out_ref):
    i = pl.program_id(0)

    @pl.when(i < NBLK)
    def _():
        off = pl.multiple_of(i * RBLK, RBLK)
        at = xt_ref[:, pl.ds(off, RBLK)].T
        out_ref[...] = jnp.concatenate([at[:RHALF], at[RHALF:]], axis=1)

    @pl.when(i == NBLK)
    def _():
        at = xt_ref[:, MAIN:NUM_SAMPLES].T
        paired = jnp.concatenate([at[:REMH], at[REMH:]], axis=1)
        pad = jnp.zeros((RHALF - REMH, NF2), jnp.float32)
        out_ref[...] = jnp.concatenate([paired, pad], axis=0)


_rp_call = pl.pallas_call(
    _rp_body,
    grid=(NBLK + 1,),
    in_specs=[pl.BlockSpec((NUM_FEATURES, NUM_SAMPLES), lambda i: (0, 0))],
    out_specs=pl.BlockSpec((RHALF, NF2), lambda i: (i, 0)),
    out_shape=jax.ShapeDtypeStruct((NWIDE, NF2), jnp.float32),
)


def _sc_body(wide_hbm, labflat_hbm, idxflat_hbm, bank_out, cnt_out, tgt_out,
             idxl_v, idxr_v, lft_v, rgt_v, ones_v, zb_v, zc_v, qidx_v, tgt_v,
             bank_sh, cnt_sh, lsem, ssem, gsem):
    cid = lax.axis_index("c")
    sid = lax.axis_index("s")
    wid = sid * NC + cid

    def load(t, b):
        j = wid + NW * t
        blk = j // CPB
        off = (j % CPB) * CW
        w0 = blk * RHALF + off
        lL = blk * RBLK + off
        lR = lL + RHALF
        fl = pltpu.async_copy(wide_hbm.at[pl.ds(w0, CW), 0:NUM_FEATURES],
                              lft_v.at[b], lsem[b])
        fr = pltpu.async_copy(wide_hbm.at[pl.ds(w0, CW), NUM_FEATURES:NF2],
                              rgt_v.at[b], lsem[b])
        il = pltpu.async_copy(labflat_hbm.at[pl.ds(lL, CW)],
                              idxl_v.at[b], lsem[b])
        ir = pltpu.async_copy(labflat_hbm.at[pl.ds(lR, CW)],
                              idxr_v.at[b], lsem[b])
        return (fl, fr, il, ir)

    ld = [None, None]
    ld[0] = load(0, 0)

    zv = jnp.zeros((16,), jnp.float32)
    ov = jnp.ones((16,), jnp.float32)

    def _fill_z(r, _):
        for k in range(NUM_FEATURES // 16):
            zb_v[r, pl.ds(k * 16, 16)] = zv
        zc_v[r, pl.ds(0, 16)] = zv
        return 0

    lax.fori_loop(0, C_PAD // NS, _fill_z, 0)
    for r in range(SUB):
        ones_v[r, pl.ds(0, 16)] = ov
    stripe = pl.ds(sid * (C_PAD // NS), C_PAD // NS)
    pltpu.sync_copy(zb_v, bank_sh.at[stripe, :])
    pltpu.sync_copy(zc_v, cnt_sh.at[stripe, :])

    pltpu.sync_copy(idxflat_hbm.at[pl.ds(wid * QPW, QPW)], qidx_v)
    pltpu.async_copy(labflat_hbm.at[qidx_v], tgt_v, gsem).wait()
    pltpu.sync_copy(tgt_v, tgt_out.at[pl.ds(wid * QPW, QPW)])

    plsc.subcore_barrier()

    def fire_scatters(b):
        out = []
        for r in range(NSUB):
            s = pl.ds(r * SUB, SUB)
            out.append(
                pltpu.async_copy(lft_v.at[b, s, :],
                                 bank_sh.at[idxl_v.at[b, s]], ssem[b],
                                 add=True))
            out.append(
                pltpu.async_copy(rgt_v.at[b, s, :],
                                 bank_sh.at[idxr_v.at[b, s]], ssem[b],
                                 add=True))
            out.append(
                pltpu.async_copy(ones_v, cnt_sh.at[idxl_v.at[b, s]],
                                 ssem[b], add=True))
            out.append(
                pltpu.async_copy(ones_v, cnt_sh.at[idxr_v.at[b, s]],
                                 ssem[b], add=True))
        return out

    scat = [[], []]
    for t in range(6):
        b = t & 1
        for d in ld[b]:
            d.wait()
        scat[b] = fire_scatters(b)
        nb = 1 - b
        for d in scat[nb]:
            d.wait()
        scat[nb] = []
        if t < 5:
            ld[nb] = load(t + 1, nb)
    for d in scat[1]:
        d.wait()

    for widx, chunks in TAIL.items():
        @pl.when(wid == widx)
        def _(chunks=chunks):
            for dw, n in chunks:
                fl = pltpu.async_copy(
                    wide_hbm.at[pl.ds(NBLK * RHALF + dw, n), 0:NUM_FEATURES],
                    lft_v.at[0, pl.ds(0, n), :], lsem[0])
                fr = pltpu.async_copy(
                    wide_hbm.at[pl.ds(NBLK * RHALF + dw, n), NUM_FEATURES:NF2],
                    rgt_v.at[0, pl.ds(0, n), :], lsem[0])
                il = pltpu.async_copy(labflat_hbm.at[pl.ds(MAIN + dw, n)],
                                      idxl_v.at[0, pl.ds(0, n)], lsem[0])
                ir = pltpu.async_copy(labflat_hbm.at[pl.ds(MAIN + REMH + dw, n)],
                                      idxr_v.at[0, pl.ds(0, n)], lsem[0])
                for d in (fl, fr, il, ir):
                    d.wait()
                ds_ = []
                s0 = 0
                for sz in TAIL_SUBS[n]:
                    s = pl.ds(s0, sz)
                    ds_.append(pltpu.async_copy(
                        lft_v.at[0, s, :], bank_sh.at[idxl_v.at[0, s]],
                        ssem[0], add=True))
                    ds_.append(pltpu.async_copy(
                        rgt_v.at[0, s, :], bank_sh.at[idxr_v.at[0, s]],
                        ssem[0], add=True))
                    ds_.append(pltpu.async_copy(
                        ones_v.at[pl.ds(0, sz), :], cnt_sh.at[idxl_v.at[0, s]],
                        ssem[0], add=True))
                    ds_.append(pltpu.async_copy(
                        ones_v.at[pl.ds(0, sz), :], cnt_sh.at[idxr_v.at[0, s]],
                        ssem[0], add=True))
                    s0 += sz
                for d in ds_:
                    d.wait()

    plsc.subcore_barrier()

    @pl.when(sid == 0)
    def _():
        pltpu.sync_copy(bank_sh, bank_out.at[cid])
        pltpu.sync_copy(cnt_sh, cnt_out.at[cid])


_sc_call = functools.partial(
    pl.kernel,
    out_type=(
        jax.ShapeDtypeStruct((NC, C_PAD, NUM_FEATURES), jnp.float32),
        jax.ShapeDtypeStruct((NC, C_PAD, 16), jnp.float32),
        jax.ShapeDtypeStruct((BATCH,), jnp.int32),
    ),
    mesh=plsc.VectorSubcoreMesh(core_axis_name="c", subcore_axis_name="s"),
    compiler_params=pltpu.CompilerParams(use_tc_tiling_on_sc=False),
    scratch_types=(
        pltpu.VMEM((2, CW), jnp.int32),
        pltpu.VMEM((2, CW), jnp.int32),
        pltpu.VMEM((2, CW, NUM_FEATURES), jnp.float32),
        pltpu.VMEM((2, CW, NUM_FEATURES), jnp.float32),
        pltpu.VMEM((SUB, 16), jnp.float32),
        pltpu.VMEM((C_PAD // NS, NUM_FEATURES), jnp.float32),
        pltpu.VMEM((C_PAD // NS, 16), jnp.float32),
        pltpu.VMEM((QPW,), jnp.int32),
        pltpu.VMEM((QPW,), jnp.int32),
        pltpu.VMEM_SHARED((C_PAD, NUM_FEATURES), jnp.float32),
        pltpu.VMEM_SHARED((C_PAD, 16), jnp.float32),
        (pltpu.SemaphoreType.DMA, pltpu.SemaphoreType.DMA),
        (pltpu.SemaphoreType.DMA, pltpu.SemaphoreType.DMA),
        pltpu.SemaphoreType.DMA,
    ),
)(_sc_body)


def _tc_body(x_ref, bank_ref, cnt_ref, tgt_ref, out_ref):
    x = x_ref[...]
    bank = bank_ref[0] + bank_ref[1]
    cnt = cnt_ref[0, :, 0:1] + cnt_ref[1, :, 0:1]
    dots = lax.dot_general(x, bank, (((0,), (1,)), ((), ())),
                           preferred_element_type=jnp.float32,
                           precision=lax.Precision.HIGHEST)
    denom = jnp.where(cnt > 0.0, cnt, 1.0)
    scale = (1.0 / TEMP) / denom
    vec = dots * scale.T
    mask = (cnt > 0.0).astype(jnp.float32).T
    exps = jnp.exp(vec) * mask
    sums = jnp.sum(exps, axis=1, keepdims=True) + 1e-6
    cids = lax.broadcasted_iota(jnp.int32, exps.shape, 1)
    texp = jnp.sum(jnp.where(cids == tgt_ref[...], exps, 0.0),
                   axis=1, keepdims=True)
    logp = jnp.log(texp / sums + 1e-6)
    out_ref[...] = -jnp.sum(logp, axis=0, keepdims=True) / float(BATCH)


_tc_call = pl.pallas_call(
    _tc_body,
    out_shape=jax.ShapeDtypeStruct((1, 1), jnp.float32),
)


def kernel(inputs, indexes, features, label):
    wide = _rp_call(features.T)
    bank2, cnt2, tgt = _sc_call(wide, label, indexes)
    loss = _tc_call(inputs.T, bank2, cnt2, tgt.reshape(BATCH, 1))
    return loss.reshape(())


# speedup vs baseline: 1.6527x; 1.0880x over previous
import functools

import jax
import jax.numpy as jnp
from jax import lax
from jax.experimental import pallas as pl
from jax.experimental.pallas import tpu as pltpu
from jax.experimental.pallas import tpu_sc as plsc

NUM_SAMPLES = 100000
NUM_FEATURES = 64
NF2 = 2 * NUM_FEATURES
NUM_CLUSTERS = 2000
C_PAD = 2048
BATCH = 1024
TEMP = 0.05

NC = 2
NS = 16
NW = NC * NS

RBLK = 2048
RHALF = RBLK // 2
NBLK = 48
MAIN = NBLK * RBLK
REM = NUM_SAMPLES - MAIN
REMH = REM // 2
NWIDE = (NBLK + 1) * RHALF

CW = 256
NCHUNK = NBLK * RHALF // CW
CPB = RHALF // CW
SUB = 128
NSUB = CW // SUB
QPW = BATCH // NW

TAIL = {0: ((0, 224), (224, 200)), 1: ((424, 224), (648, 200))}
TAIL_SUBS = {224: (128, 96), 200: (128, 72)}


def _rp_body(xt_ref, out_ref):
    i = pl.program_id(0)

    def pair(off):
        at = xt_ref[:, pl.ds(pl.multiple_of(off, RBLK), RBLK)].T
        return jnp.concatenate([at[:RHALF], at[RHALF:]], axis=1)

    @pl.when(i < NBLK // 2)
    def _():
        out_ref[...] = jnp.concatenate(
            [pair(2 * i * RBLK), pair((2 * i + 1) * RBLK)], axis=0)

    @pl.when(i == NBLK // 2)
    def _():
        at = xt_ref[:, MAIN:NUM_SAMPLES].T
        paired = jnp.concatenate([at[:REMH], at[REMH:]], axis=1)
        pad = jnp.zeros((2 * RHALF - REMH, NF2), jnp.float32)
        out_ref[...] = jnp.concatenate([paired, pad], axis=0)


_rp_call = pl.pallas_call(
    _rp_body,
    grid=(NBLK // 2 + 1,),
    in_specs=[pl.BlockSpec((NUM_FEATURES, NUM_SAMPLES), lambda i: (0, 0))],
    out_specs=pl.BlockSpec((2 * RHALF, NF2), lambda i: (i, 0)),
    out_shape=jax.ShapeDtypeStruct((NWIDE + RHALF, NF2), jnp.float32),
)


def _sc_body(wide_hbm, labflat_hbm, idxflat_hbm, bank_out, cnt_out, tgt_out,
             idxl_v, idxr_v, lft_v, rgt_v, ones_v, zb_v, zc_v, qidx_v, tgt_v,
             bank_sh, cnt_sh, lsem, ssem, gsem):
    cid = lax.axis_index("c")
    sid = lax.axis_index("s")
    wid = sid * NC + cid

    def load(t, b):
        j = wid + NW * t
        blk = j // CPB
        off = (j % CPB) * CW
        w0 = blk * RHALF + off
        lL = blk * RBLK + off
        lR = lL + RHALF
        fl = pltpu.async_copy(wide_hbm.at[pl.ds(w0, CW), 0:NUM_FEATURES],
                              lft_v.at[b], lsem[b])
        fr = pltpu.async_copy(wide_hbm.at[pl.ds(w0, CW), NUM_FEATURES:NF2],
                              rgt_v.at[b], lsem[b])
        il = pltpu.async_copy(labflat_hbm.at[pl.ds(lL, CW)],
                              idxl_v.at[b], lsem[b])
        ir = pltpu.async_copy(labflat_hbm.at[pl.ds(lR, CW)],
                              idxr_v.at[b], lsem[b])
        return (fl, fr, il, ir)

    ld = [None, None]
    ld[0] = load(0, 0)

    zv = jnp.zeros((16,), jnp.float32)
    ov = jnp.ones((16,), jnp.float32)

    def _fill_z(r, _):
        for k in range(NUM_FEATURES // 16):
            zb_v[r, pl.ds(k * 16, 16)] = zv
        zc_v[r, pl.ds(0, 16)] = zv
        return 0

    lax.fori_loop(0, C_PAD // NS, _fill_z, 0)
    for r in range(SUB):
        ones_v[r, pl.ds(0, 16)] = ov
    stripe = pl.ds(sid * (C_PAD // NS), C_PAD // NS)
    pltpu.sync_copy(zb_v, bank_sh.at[stripe, :])
    pltpu.sync_copy(zc_v, cnt_sh.at[stripe, :])

    pltpu.sync_copy(idxflat_hbm.at[pl.ds(wid * QPW, QPW)], qidx_v)
    pltpu.async_copy(labflat_hbm.at[qidx_v], tgt_v, gsem).wait()
    pltpu.sync_copy(tgt_v, tgt_out.at[pl.ds(wid * QPW, QPW)])

    plsc.subcore_barrier()

    def fire_scatters(b):
        out = []
        for r in range(NSUB):
            s = pl.ds(r * SUB, SUB)
            out.append(
                pltpu.async_copy(lft_v.at[b, s, :],
                                 bank_sh.at[idxl_v.at[b, s]], ssem[b],
                                 add=True))
            out.append(
                pltpu.async_copy(rgt_v.at[b, s, :],
                                 bank_sh.at[idxr_v.at[b, s]], ssem[b],
                                 add=True))
            out.append(
                pltpu.async_copy(ones_v, cnt_sh.at[idxl_v.at[b, s]],
                                 ssem[b], add=True))
            out.append(
                pltpu.async_copy(ones_v, cnt_sh.at[idxr_v.at[b, s]],
                                 ssem[b], add=True))
        return out

    scat = [[], []]
    for t in range(6):
        b = t & 1
        for d in ld[b]:
            d.wait()
        scat[b] = fire_scatters(b)
        nb = 1 - b
        for d in scat[nb]:
            d.wait()
        scat[nb] = []
        if t < 5:
            ld[nb] = load(t + 1, nb)
    for d in scat[1]:
        d.wait()

    for widx, chunks in TAIL.items():
        @pl.when(wid == widx)
        def _(chunks=chunks):
            for dw, n in chunks:
                fl = pltpu.async_copy(
                    wide_hbm.at[pl.ds(NBLK * RHALF + dw, n), 0:NUM_FEATURES],
                    lft_v.at[0, pl.ds(0, n), :], lsem[0])
                fr = pltpu.async_copy(
                    wide_hbm.at[pl.ds(NBLK * RHALF + dw, n), NUM_FEATURES:NF2],
                    rgt_v.at[0, pl.ds(0, n), :], lsem[0])
                il = pltpu.async_copy(labflat_hbm.at[pl.ds(MAIN + dw, n)],
                                      idxl_v.at[0, pl.ds(0, n)], lsem[0])
                ir = pltpu.async_copy(labflat_hbm.at[pl.ds(MAIN + REMH + dw, n)],
                                      idxr_v.at[0, pl.ds(0, n)], lsem[0])
                for d in (fl, fr, il, ir):
                    d.wait()
                ds_ = []
                s0 = 0
                for sz in TAIL_SUBS[n]:
                    s = pl.ds(s0, sz)
                    ds_.append(pltpu.async_copy(
                        lft_v.at[0, s, :], bank_sh.at[idxl_v.at[0, s]],
                        ssem[0], add=True))
                    ds_.append(pltpu.async_copy(
                        rgt_v.at[0, s, :], bank_sh.at[idxr_v.at[0, s]],
                        ssem[0], add=True))
                    ds_.append(pltpu.async_copy(
                        ones_v.at[pl.ds(0, sz), :], cnt_sh.at[idxl_v.at[0, s]],
                        ssem[0], add=True))
                    ds_.append(pltpu.async_copy(
                        ones_v.at[pl.ds(0, sz), :], cnt_sh.at[idxr_v.at[0, s]],
                        ssem[0], add=True))
                    s0 += sz
                for d in ds_:
                    d.wait()

    plsc.subcore_barrier()

    @pl.when(sid == 0)
    def _():
        pltpu.sync_copy(bank_sh, bank_out.at[cid])
        pltpu.sync_copy(cnt_sh, cnt_out.at[cid])


_sc_call = functools.partial(
    pl.kernel,
    out_type=(
        jax.ShapeDtypeStruct((NC, C_PAD, NUM_FEATURES), jnp.float32),
        jax.ShapeDtypeStruct((NC, C_PAD, 16), jnp.float32),
        jax.ShapeDtypeStruct((BATCH,), jnp.int32),
    ),
    mesh=plsc.VectorSubcoreMesh(core_axis_name="c", subcore_axis_name="s"),
    compiler_params=pltpu.CompilerParams(use_tc_tiling_on_sc=False),
    scratch_types=(
        pltpu.VMEM((2, CW), jnp.int32),
        pltpu.VMEM((2, CW), jnp.int32),
        pltpu.VMEM((2, CW, NUM_FEATURES), jnp.float32),
        pltpu.VMEM((2, CW, NUM_FEATURES), jnp.float32),
        pltpu.VMEM((SUB, 16), jnp.float32),
        pltpu.VMEM((C_PAD // NS, NUM_FEATURES), jnp.float32),
        pltpu.VMEM((C_PAD // NS, 16), jnp.float32),
        pltpu.VMEM((QPW,), jnp.int32),
        pltpu.VMEM((QPW,), jnp.int32),
        pltpu.VMEM_SHARED((C_PAD, NUM_FEATURES), jnp.float32),
        pltpu.VMEM_SHARED((C_PAD, 16), jnp.float32),
        (pltpu.SemaphoreType.DMA, pltpu.SemaphoreType.DMA),
        (pltpu.SemaphoreType.DMA, pltpu.SemaphoreType.DMA),
        pltpu.SemaphoreType.DMA,
    ),
)(_sc_body)


def _tc_body(x_ref, bank_ref, cnt_ref, tgt_ref, out_ref):
    x = x_ref[...]
    bank = bank_ref[0] + bank_ref[1]
    cnt = cnt_ref[0, :, 0:1] + cnt_ref[1, :, 0:1]
    dots = lax.dot_general(x, bank, (((0,), (1,)), ((), ())),
                           preferred_element_type=jnp.float32,
                           precision=lax.Precision.HIGHEST)
    denom = jnp.where(cnt > 0.0, cnt, 1.0)
    scale = (1.0 / TEMP) / denom
    vec = dots * scale.T
    mask = (cnt > 0.0).astype(jnp.float32).T
    exps = jnp.exp(vec) * mask
    sums = jnp.sum(exps, axis=1, keepdims=True) + 1e-6
    cids = lax.broadcasted_iota(jnp.int32, exps.shape, 1)
    texp = jnp.sum(jnp.where(cids == tgt_ref[...], exps, 0.0),
                   axis=1, keepdims=True)
    logp = jnp.log(texp / sums + 1e-6)
    out_ref[...] = -jnp.sum(logp, axis=0, keepdims=True) / float(BATCH)


_tc_call = pl.pallas_call(
    _tc_body,
    out_shape=jax.ShapeDtypeStruct((1, 1), jnp.float32),
)


def kernel(inputs, indexes, features, label):
    wide = _rp_call(features.T)
    bank2, cnt2, tgt = _sc_call(wide, label, indexes)
    loss = _tc_call(inputs.T, bank2, cnt2, tgt.reshape(BATCH, 1))
    return loss.reshape(())


# speedup vs baseline: 1.6769x; 1.0146x over previous
import functools

import jax
import jax.numpy as jnp
from jax import lax
from jax.experimental import pallas as pl
from jax.experimental.pallas import tpu as pltpu
from jax.experimental.pallas import tpu_sc as plsc

NUM_SAMPLES = 100000
NUM_FEATURES = 64
NF2 = 2 * NUM_FEATURES
NUM_CLUSTERS = 2000
C_PAD = 2048
BATCH = 1024
TEMP = 0.05

NC = 2
NS = 16
NW = NC * NS

RBLK = 2048
RHALF = RBLK // 2
NBLK = 48
MAIN = NBLK * RBLK
REM = NUM_SAMPLES - MAIN
REMH = REM // 2
NWIDE = (NBLK + 1) * RHALF

CW = 256
NCHUNK = NBLK * RHALF // CW
CPB = RHALF // CW
SUB = 128
NSUB = CW // SUB
QPW = BATCH // NW

TAIL = {0: ((0, 224), (224, 200)), 1: ((424, 224), (648, 200))}
TAIL_SUBS = {224: (128, 96), 200: (128, 72)}


def _rp_body(xt_ref, out_ref):
    i = pl.program_id(0)

    def pair(off):
        at = xt_ref[:, pl.ds(pl.multiple_of(off, RBLK), RBLK)].T
        return jnp.concatenate([at[:RHALF], at[RHALF:]], axis=1)

    @pl.when(i < NBLK // 4)
    def _():
        out_ref[...] = jnp.concatenate(
            [pair((4 * i + k) * RBLK) for k in range(4)], axis=0)

    @pl.when(i == NBLK // 4)
    def _():
        at = xt_ref[:, MAIN:NUM_SAMPLES].T
        paired = jnp.concatenate([at[:REMH], at[REMH:]], axis=1)
        pad = jnp.zeros((4 * RHALF - REMH, NF2), jnp.float32)
        out_ref[...] = jnp.concatenate([paired, pad], axis=0)


_rp_call = pl.pallas_call(
    _rp_body,
    grid=(NBLK // 4 + 1,),
    in_specs=[pl.BlockSpec((NUM_FEATURES, NUM_SAMPLES), lambda i: (0, 0))],
    out_specs=pl.BlockSpec((4 * RHALF, NF2), lambda i: (i, 0)),
    out_shape=jax.ShapeDtypeStruct((NWIDE + 3 * RHALF, NF2), jnp.float32),
)


def _sc_body(wide_hbm, labflat_hbm, idxflat_hbm, bank_out, cnt_out, tgt_out,
             idxl_v, idxr_v, lft_v, rgt_v, ones_v, zb_v, zc_v, qidx_v, tgt_v,
             bank_sh, cnt_sh, lsem, ssem, gsem):
    cid = lax.axis_index("c")
    sid = lax.axis_index("s")
    wid = sid * NC + cid

    def load(t, b):
        j = wid + NW * t
        blk = j // CPB
        off = (j % CPB) * CW
        w0 = blk * RHALF + off
        lL = blk * RBLK + off
        lR = lL + RHALF
        fl = pltpu.async_copy(wide_hbm.at[pl.ds(w0, CW), 0:NUM_FEATURES],
                              lft_v.at[b], lsem[b])
        fr = pltpu.async_copy(wide_hbm.at[pl.ds(w0, CW), NUM_FEATURES:NF2],
                              rgt_v.at[b], lsem[b])
        il = pltpu.async_copy(labflat_hbm.at[pl.ds(lL, CW)],
                              idxl_v.at[b], lsem[b])
        ir = pltpu.async_copy(labflat_hbm.at[pl.ds(lR, CW)],
                              idxr_v.at[b], lsem[b])
        return (fl, fr, il, ir)

    ld = [None, None]
    ld[0] = load(0, 0)

    zv = jnp.zeros((16,), jnp.float32)
    ov = jnp.ones((16,), jnp.float32)

    def _fill_z(r, _):
        for k in range(NUM_FEATURES // 16):
            zb_v[r, pl.ds(k * 16, 16)] = zv
        zc_v[r, pl.ds(0, 16)] = zv
        return 0

    lax.fori_loop(0, C_PAD // NS, _fill_z, 0)
    for r in range(SUB):
        ones_v[r, pl.ds(0, 16)] = ov
    stripe = pl.ds(sid * (C_PAD // NS), C_PAD // NS)
    pltpu.sync_copy(zb_v, bank_sh.at[stripe, :])
    pltpu.sync_copy(zc_v, cnt_sh.at[stripe, :])

    pltpu.sync_copy(idxflat_hbm.at[pl.ds(wid * QPW, QPW)], qidx_v)
    pltpu.async_copy(labflat_hbm.at[qidx_v], tgt_v, gsem).wait()
    pltpu.sync_copy(tgt_v, tgt_out.at[pl.ds(wid * QPW, QPW)])

    plsc.subcore_barrier()

    def fire_scatters(b):
        out = []
        for r in range(NSUB):
            s = pl.ds(r * SUB, SUB)
            out.append(
                pltpu.async_copy(lft_v.at[b, s, :],
                                 bank_sh.at[idxl_v.at[b, s]], ssem[b],
                                 add=True))
            out.append(
                pltpu.async_copy(rgt_v.at[b, s, :],
                                 bank_sh.at[idxr_v.at[b, s]], ssem[b],
                                 add=True))
            out.append(
                pltpu.async_copy(ones_v, cnt_sh.at[idxl_v.at[b, s]],
                                 ssem[b], add=True))
            out.append(
                pltpu.async_copy(ones_v, cnt_sh.at[idxr_v.at[b, s]],
                                 ssem[b], add=True))
        return out

    scat = [[], []]
    for t in range(6):
        b = t & 1
        for d in ld[b]:
            d.wait()
        scat[b] = fire_scatters(b)
        nb = 1 - b
        for d in scat[nb]:
            d.wait()
        scat[nb] = []
        if t < 5:
            ld[nb] = load(t + 1, nb)
    for d in scat[1]:
        d.wait()

    for widx, chunks in TAIL.items():
        @pl.when(wid == widx)
        def _(chunks=chunks):
            for dw, n in chunks:
                fl = pltpu.async_copy(
                    wide_hbm.at[pl.ds(NBLK * RHALF + dw, n), 0:NUM_FEATURES],
                    lft_v.at[0, pl.ds(0, n), :], lsem[0])
                fr = pltpu.async_copy(
                    wide_hbm.at[pl.ds(NBLK * RHALF + dw, n), NUM_FEATURES:NF2],
                    rgt_v.at[0, pl.ds(0, n), :], lsem[0])
                il = pltpu.async_copy(labflat_hbm.at[pl.ds(MAIN + dw, n)],
                                      idxl_v.at[0, pl.ds(0, n)], lsem[0])
                ir = pltpu.async_copy(labflat_hbm.at[pl.ds(MAIN + REMH + dw, n)],
                                      idxr_v.at[0, pl.ds(0, n)], lsem[0])
                for d in (fl, fr, il, ir):
                    d.wait()
                ds_ = []
                s0 = 0
                for sz in TAIL_SUBS[n]:
                    s = pl.ds(s0, sz)
                    ds_.append(pltpu.async_copy(
                        lft_v.at[0, s, :], bank_sh.at[idxl_v.at[0, s]],
                        ssem[0], add=True))
                    ds_.append(pltpu.async_copy(
                        rgt_v.at[0, s, :], bank_sh.at[idxr_v.at[0, s]],
                        ssem[0], add=True))
                    ds_.append(pltpu.async_copy(
                        ones_v.at[pl.ds(0, sz), :], cnt_sh.at[idxl_v.at[0, s]],
                        ssem[0], add=True))
                    ds_.append(pltpu.async_copy(
                        ones_v.at[pl.ds(0, sz), :], cnt_sh.at[idxr_v.at[0, s]],
                        ssem[0], add=True))
                    s0 += sz
                for d in ds_:
                    d.wait()

    plsc.subcore_barrier()

    @pl.when(sid == 0)
    def _():
        pltpu.sync_copy(bank_sh, bank_out.at[cid])
        pltpu.sync_copy(cnt_sh, cnt_out.at[cid])


_sc_call = functools.partial(
    pl.kernel,
    out_type=(
        jax.ShapeDtypeStruct((NC, C_PAD, NUM_FEATURES), jnp.float32),
        jax.ShapeDtypeStruct((NC, C_PAD, 16), jnp.float32),
        jax.ShapeDtypeStruct((BATCH,), jnp.int32),
    ),
    mesh=plsc.VectorSubcoreMesh(core_axis_name="c", subcore_axis_name="s"),
    compiler_params=pltpu.CompilerParams(use_tc_tiling_on_sc=False),
    scratch_types=(
        pltpu.VMEM((2, CW), jnp.int32),
        pltpu.VMEM((2, CW), jnp.int32),
        pltpu.VMEM((2, CW, NUM_FEATURES), jnp.float32),
        pltpu.VMEM((2, CW, NUM_FEATURES), jnp.float32),
        pltpu.VMEM((SUB, 16), jnp.float32),
        pltpu.VMEM((C_PAD // NS, NUM_FEATURES), jnp.float32),
        pltpu.VMEM((C_PAD // NS, 16), jnp.float32),
        pltpu.VMEM((QPW,), jnp.int32),
        pltpu.VMEM((QPW,), jnp.int32),
        pltpu.VMEM_SHARED((C_PAD, NUM_FEATURES), jnp.float32),
        pltpu.VMEM_SHARED((C_PAD, 16), jnp.float32),
        (pltpu.SemaphoreType.DMA, pltpu.SemaphoreType.DMA),
        (pltpu.SemaphoreType.DMA, pltpu.SemaphoreType.DMA),
        pltpu.SemaphoreType.DMA,
    ),
)(_sc_body)


def _tc_body(x_ref, bank_ref, cnt_ref, tgt_ref, out_ref):
    x = x_ref[...]
    bank = bank_ref[0] + bank_ref[1]
    cnt = cnt_ref[0, :, 0:1] + cnt_ref[1, :, 0:1]
    dots = lax.dot_general(x, bank, (((0,), (1,)), ((), ())),
                           preferred_element_type=jnp.float32,
                           precision=lax.Precision.HIGHEST)
    denom = jnp.where(cnt > 0.0, cnt, 1.0)
    scale = (1.0 / TEMP) / denom
    vec = dots * scale.T
    mask = (cnt > 0.0).astype(jnp.float32).T
    exps = jnp.exp(vec) * mask
    sums = jnp.sum(exps, axis=1, keepdims=True) + 1e-6
    cids = lax.broadcasted_iota(jnp.int32, exps.shape, 1)
    texp = jnp.sum(jnp.where(cids == tgt_ref[...], exps, 0.0),
                   axis=1, keepdims=True)
    logp = jnp.log(texp / sums + 1e-6)
    out_ref[...] = -jnp.sum(logp, axis=0, keepdims=True) / float(BATCH)


_tc_call = pl.pallas_call(
    _tc_body,
    out_shape=jax.ShapeDtypeStruct((1, 1), jnp.float32),
)


def kernel(inputs, indexes, features, label):
    wide = _rp_call(features.T)
    bank2, cnt2, tgt = _sc_call(wide, label, indexes)
    loss = _tc_call(inputs.T, bank2, cnt2, tgt.reshape(BATCH, 1))
    return loss.reshape(())
